# baseline (speedup 1.0000x reference)
import functools

import jax
import jax.numpy as jnp
from jax import lax
from jax.experimental import pallas as pl
from jax.experimental.pallas import tpu as pltpu
from jax.experimental.pallas import tpu_sc as plsc

B = 64
NB = 4
V = 100000
EOS = 2
REP = 1.2
BN = B * NB

NC, NS, LANES = 2, 16, 16
NW = NC * NS
RPT = BN // NW
BC = 4096
NFULL = 24
SLABS = 1664
VTAIL = V - NFULL * BC - SLABS
TOKP = 64
TOKVECS = TOKP // LANES
CANDW = 384
K = 2 * NB
NEG = -3.0e38
BIGI = 2**30


def _iota16(off):
    return lax.iota(jnp.int32, 16) + off


def _store1(ref, pos, val):
    lane0 = lax.iota(jnp.int32, LANES) == 0
    plsc.store_scatter(ref, [jnp.broadcast_to(pos, (LANES,))],
                       jnp.broadcast_to(val, (LANES,)), mask=lane0)


def _load1(ref, pos):
    g = plsc.load_gather(ref, [jnp.broadcast_to(pos, (LANES,))])
    return jnp.max(g)


def _sc_stage(scores, scores_tail, tokens_p):
    mesh = plsc.VectorSubcoreMesh(core_axis_name="c", subcore_axis_name="s")

    @functools.partial(
        pl.kernel,
        out_type=(
            jax.ShapeDtypeStruct((BN * K,), jnp.float32),
            jax.ShapeDtypeStruct((BN * K,), jnp.int32),
            jax.ShapeDtypeStruct((BN,), jnp.float32),
            jax.ShapeDtypeStruct((BN,), jnp.float32),
        ),
        mesh=mesh,
        compiler_params=pltpu.CompilerParams(needs_layout_passes=False),
        scratch_types=[
            pltpu.VMEM((RPT, BC), jnp.float32),
            pltpu.VMEM((RPT, BC), jnp.float32),
            pltpu.VMEM((RPT, SLABS), jnp.float32),
            pltpu.VMEM((RPT, VTAIL), jnp.float32),
            pltpu.VMEM((RPT, TOKP), jnp.int32),
            pltpu.VMEM((RPT * CANDW,), jnp.float32),
            pltpu.VMEM((RPT * CANDW,), jnp.int32),
            pltpu.VMEM((LANES,), jnp.float32),
            pltpu.VMEM((LANES,), jnp.float32),
            pltpu.VMEM((LANES,), jnp.float32),
            pltpu.VMEM((RPT * LANES,), jnp.float32),
            pltpu.VMEM((LANES,), jnp.int32),
            pltpu.VMEM((RPT * K,), jnp.float32),
            pltpu.VMEM((RPT * K,), jnp.int32),
            pltpu.VMEM((RPT,), jnp.float32),
            pltpu.VMEM((RPT,), jnp.float32),
            pltpu.SemaphoreType.DMA,
            pltpu.SemaphoreType.DMA,
        ],
    )
    def sc_kernel(scores_hbm, tail_hbm, tok_hbm, ov_hbm, oc_hbm, om_hbm,
                  os_hbm, bufA, bufB, bufS, bufU, tokbuf, candv, candc, srt,
                  mrow, trow, srow, cntrow, t8v, t8c, m8, s8, semA, semB):
        wid = lax.axis_index("s") * NC + lax.axis_index("c")
        r0 = wid * RPT

        pltpu.sync_copy(tok_hbm.at[pl.ds(r0, RPT), :], tokbuf)

        def cinit(i, _):
            candv[pl.ds(i * LANES, LANES)] = jnp.full((LANES,), NEG, jnp.float32)
            return 0
        lax.fori_loop(0, RPT * CANDW // LANES, cinit, 0)
        mrow[...] = jnp.full((LANES,), NEG, jnp.float32)
        trow[...] = jnp.full((LANES,), NEG, jnp.float32)
        cntrow[...] = jnp.zeros((LANES,), jnp.int32)
        for r in range(RPT):
            srow[pl.ds(r * LANES, LANES)] = jnp.zeros((LANES,), jnp.float32)

        def start(c, dst, sem, w):
            pltpu.async_copy(
                scores_hbm.at[pl.ds(r0, RPT), pl.ds(c * BC, w)], dst, sem)

        def wait(dst, sem, w):
            pltpu.make_async_copy(
                scores_hbm.at[pl.ds(0, RPT), pl.ds(0, w)], dst, sem).wait()

        def penalty(sbuf, r, c0, w):
            for t in range(TOKVECS):
                tok = tokbuf[r, pl.ds(t * LANES, LANES)]
                loc = tok - c0
                msk = (loc >= 0) & (loc < w)
                locc = jnp.where(msk, loc, 0)
                rsp = jnp.broadcast_to(r, (LANES,))
                g = plsc.load_gather(sbuf, [rsp, locc], mask=msk)
                pen = jnp.where(g < 0.0, g * REP, g * (1.0 / REP))
                plsc.store_scatter(sbuf, [rsp, locc], pen, mask=msk)

        def put_hits(x, msk, cbase, cnt, col0):
            cnt = jnp.minimum(cnt, CANDW - LANES)
            plsc.store_compressed(candv.at[pl.ds(cbase + cnt, LANES)], x,
                                  mask=msk)
            plsc.store_compressed(candc.at[pl.ds(cbase + cnt, LANES)],
                                  _iota16(col0), mask=msk)
            return cnt + jnp.sum(msk.astype(jnp.int32))

        def expsum(sbuf, r, nv, mbc):
            m_old = _load1(mrow, r)
            m_new = jnp.maximum(m_old, mbc)
            s_run = (srow[pl.ds(r * LANES, LANES)]
                     * jnp.exp(jnp.broadcast_to(m_old - m_new, (LANES,))))

            def bsum(i, acc):
                x = sbuf[r, pl.ds(i * LANES, LANES)]
                return acc + jnp.exp(x - m_new)
            s_run = lax.fori_loop(0, nv, bsum, s_run, unroll=8)
            srow[pl.ds(r * LANES, LANES)] = s_run
            _store1(mrow, r, m_new)

        def t_of(lmax):
            srt[...] = plsc.sort_key_val(lmax, lmax, descending=True)[0]
            return _load1(srt, K - 1)

        def process0(sbuf, c0, w):
            nv = w // LANES

            def rbody(r, _):
                penalty(sbuf, r, c0, w)

                def bmax(i, mv):
                    return jnp.maximum(mv, sbuf[r, pl.ds(i * LANES, LANES)])
                lmax = lax.fori_loop(0, nv, bmax,
                                     jnp.full((LANES,), NEG, jnp.float32),
                                     unroll=8)
                t_loc = t_of(lmax)
                _store1(trow, r, t_loc)
                cbase = r * CANDW

                def bcol(i, cnt):
                    x = sbuf[r, pl.ds(i * LANES, LANES)]
                    msk = x >= t_loc
                    npos = jnp.sum(msk.astype(jnp.int32))
                    return lax.cond(
                        npos > 0,
                        lambda n: put_hits(x, msk, cbase, n, c0 + i * LANES),
                        lambda n: n, cnt)
                cnt = lax.fori_loop(0, nv, bcol, _load1(cntrow, r))
                _store1(cntrow, r, cnt)
                expsum(sbuf, r, nv, jnp.max(lmax))
                return 0

            lax.fori_loop(0, RPT, rbody, 0)

        def process_n(sbuf, c0, w):
            nv = w // LANES

            def rbody(r, _):
                penalty(sbuf, r, c0, w)
                t_run = _load1(trow, r)
                cbase = r * CANDW

                def bmax(i, car):
                    mv, cnt = car
                    x = sbuf[r, pl.ds(i * LANES, LANES)]
                    msk = x >= t_run
                    npos = jnp.sum(msk.astype(jnp.int32))
                    cnt = lax.cond(
                        npos > 0,
                        lambda n: put_hits(x, msk, cbase, n, c0 + i * LANES),
                        lambda n: n, cnt)
                    return jnp.maximum(mv, x), cnt
                lmax, cnt = lax.fori_loop(
                    0, nv, bmax,
                    (jnp.full((LANES,), NEG, jnp.float32), _load1(cntrow, r)),
                    unroll=8)
                _store1(cntrow, r, cnt)
                t_loc = t_of(lmax)
                _store1(trow, r, jnp.maximum(t_run, t_loc))
                expsum(sbuf, r, nv, jnp.max(lmax))
                return 0

            lax.fori_loop(0, RPT, rbody, 0)

        start(0, bufA, semA, BC)
        start(1, bufB, semB, BC)
        wait(bufA, semA, BC)
        process0(bufA, 0, BC)
        start(2, bufA, semA, BC)

        def group(g, _):
            c = 2 * g + 1
            wait(bufB, semB, BC)
            start(c + 2, bufB, semB, BC)
            process_n(bufB, c * BC, BC)
            wait(bufA, semA, BC)
            lax.cond(c + 3 <= NFULL - 1,
                     lambda: start(c + 3, bufA, semA, BC), lambda: None)
            process_n(bufA, (c + 1) * BC, BC)
            return 0

        lax.fori_loop(0, (NFULL - 2) // 2, group, 0)
        wait(bufB, semB, BC)
        pltpu.async_copy(
            scores_hbm.at[pl.ds(r0, RPT), pl.ds(NFULL * BC, SLABS)],
            bufS, semA)
        process_n(bufB, (NFULL - 1) * BC, BC)
        pltpu.make_async_copy(
            scores_hbm.at[pl.ds(0, RPT), pl.ds(0, SLABS)], bufS, semA).wait()
        pltpu.async_copy(tail_hbm.at[pl.ds(r0, RPT), :], bufU, semB)
        process_n(bufS, NFULL * BC, SLABS)
        pltpu.make_async_copy(
            tail_hbm.at[pl.ds(0, RPT), :], bufU, semB).wait()
        process_n(bufU, NFULL * BC + SLABS, VTAIL)

        def drow(r, _):
            cbase = r * CANDW
            nv = (_load1(cntrow, r) + LANES - 1) >> 4

            def dstep(j, _2):
                def body(i, car):
                    mv, av = car
                    x = candv[pl.ds(cbase + i * LANES, LANES)]
                    idx = _iota16(i * LANES)
                    upd = x > mv
                    return jnp.maximum(mv, x), jnp.where(upd, idx, av)
                mv, av = lax.fori_loop(
                    0, nv, body,
                    (jnp.full((LANES,), NEG, jnp.float32),
                     jnp.full((LANES,), BIGI, jnp.int32)))
                m = jnp.max(mv)
                rel = jnp.min(jnp.where(mv >= m, av, BIGI))
                col = _load1(candc, cbase + rel)
                _store1(candv, cbase + rel, jnp.float32(NEG))
                _store1(t8v, r * K + j, m)
                _store1(t8c, r * K + j, col)
                return 0
            lax.fori_loop(0, K, dstep, 0)
            _store1(m8, r, _load1(mrow, r))
            _store1(s8, r, jnp.sum(srow[pl.ds(r * LANES, LANES)]))
            return 0
        lax.fori_loop(0, RPT, drow, 0)

        pltpu.sync_copy(t8v, ov_hbm.at[pl.ds(r0 * K, RPT * K)])
        pltpu.sync_copy(t8c, oc_hbm.at[pl.ds(r0 * K, RPT * K)])
        pltpu.sync_copy(m8, om_hbm.at[pl.ds(r0, RPT)])
        pltpu.sync_copy(s8, os_hbm.at[pl.ds(r0, RPT)])

    return sc_kernel(scores, scores_tail, tokens_p)


def _tc_merge_body(v_ref, c_ref, m_ref, s_ref, b_ref,
                   ks_ref, kt_ref, kb_ref, ri_ref):
    x = v_ref[:]
    cols = c_ref[:]
    logp = x - m_ref[:] - jnp.log(s_ref[:]) + b_ref[:]
    pos = lax.broadcasted_iota(jnp.int32, (B, NB * K), 1)

    remaining = logp
    nb_seen = jnp.zeros((B, 1), jnp.int32)
    ks = [jnp.zeros((B, 1), jnp.float32) for _ in range(NB)]
    kt = [jnp.zeros((B, 1), jnp.int32) for _ in range(NB)]
    kb = [jnp.zeros((B, 1), jnp.int32) for _ in range(NB)]
    for _ in range(K):
        mx = jnp.max(remaining, axis=1, keepdims=True)
        p = jnp.min(jnp.where(remaining >= mx, pos, BIGI), axis=1, keepdims=True)
        hit = pos == p
        tok = jnp.sum(jnp.where(hit, cols, 0), axis=1, keepdims=True)
        beam = p // K
        remaining = jnp.where(hit, NEG, remaining)
        not_eos = (tok != EOS).astype(jnp.int32)
        rank = nb_seen + not_eos
        for q in range(NB):
            keep = (not_eos == 1) & (rank == q + 1)
            ks[q] = jnp.where(keep, mx, ks[q])
            kt[q] = jnp.where(keep, tok, kt[q])
            kb[q] = jnp.where(keep, beam, kb[q])
        nb_seen = nb_seen + not_eos

    ks_ref[:] = jnp.concatenate(ks, axis=1)
    kt_ref[:] = jnp.concatenate(kt, axis=1)
    kbm = jnp.concatenate(kb, axis=1)
    kb_ref[:] = kbm
    ri_ref[:] = kbm + jnp.arange(B, dtype=jnp.int32)[:, None] * NB


def _tc_merge(vals, cols, m, s, beam_scores):
    v32 = vals.reshape(B, NB * K)
    c32 = cols.reshape(B, NB * K)
    m32 = jnp.repeat(m.reshape(B, NB), K, axis=1)
    s32 = jnp.repeat(s.reshape(B, NB), K, axis=1)
    b32 = jnp.repeat(beam_scores.reshape(B, NB), K, axis=1)
    ks, kt, kb, ri = pl.pallas_call(
        _tc_merge_body,
        out_shape=(
            jax.ShapeDtypeStruct((B, NB), jnp.float32),
            jax.ShapeDtypeStruct((B, NB), jnp.int32),
            jax.ShapeDtypeStruct((B, NB), jnp.int32),
            jax.ShapeDtypeStruct((B, NB), jnp.int32),
        ),
    )(v32, c32, m32, s32, b32)
    return ks, kt, kb, ri.reshape(-1)


def kernel(scores, beam_scores, token_ids):
    tokens_p = jnp.pad(token_ids, ((0, 0), (0, TOKP - token_ids.shape[1])),
                       constant_values=BIGI)
    scores_tail = lax.slice(scores, (0, NFULL * BC + SLABS), (BN, V))
    vals, cols, m, s = _sc_stage(scores, scores_tail, tokens_p)
    return _tc_merge(vals, cols, m, s, beam_scores)


# speedup vs baseline: 2.4089x; 2.4089x over previous
import functools

import jax
import jax.numpy as jnp
from jax import lax
from jax.experimental import pallas as pl
from jax.experimental.pallas import tpu as pltpu
from jax.experimental.pallas import tpu_sc as plsc

B = 64
NB = 4
V = 100000
EOS = 2
REP = 1.2
BN = B * NB

NC, NS, LANES = 2, 16, 16
NW = NC * NS
RPT = BN // NW
BC = 4096
NFULL = 24
SLABS = 1664
VTAIL = V - NFULL * BC - SLABS
TOKP = 64
TOKVECS = TOKP // LANES
LCAP = 48
K = 2 * NB
NEG = -3.0e38
BIGI = 2**30


def _iota16(off):
    return lax.iota(jnp.int32, 16) + off


def _store1(ref, pos, val):
    lane0 = lax.iota(jnp.int32, LANES) == 0
    plsc.store_scatter(ref, [jnp.broadcast_to(pos, (LANES,))],
                       jnp.broadcast_to(val, (LANES,)), mask=lane0)


def _load1(ref, pos):
    g = plsc.load_gather(ref, [jnp.broadcast_to(pos, (LANES,))])
    return jnp.max(g)


def _sc_stage(scores, scores_tail, tokens_p):
    mesh = plsc.VectorSubcoreMesh(core_axis_name="c", subcore_axis_name="s")

    @functools.partial(
        pl.kernel,
        out_type=(
            jax.ShapeDtypeStruct((BN * K,), jnp.float32),
            jax.ShapeDtypeStruct((BN * K,), jnp.int32),
            jax.ShapeDtypeStruct((BN,), jnp.float32),
            jax.ShapeDtypeStruct((BN,), jnp.float32),
        ),
        mesh=mesh,
        compiler_params=pltpu.CompilerParams(needs_layout_passes=False),
        scratch_types=[
            pltpu.VMEM((RPT, BC), jnp.float32),
            pltpu.VMEM((RPT, BC), jnp.float32),
            pltpu.VMEM((RPT, SLABS), jnp.float32),
            pltpu.VMEM((RPT, VTAIL), jnp.float32),
            pltpu.VMEM((RPT, TOKP), jnp.int32),
            pltpu.VMEM((RPT * LANES * LCAP,), jnp.float32),
            pltpu.VMEM((RPT * LANES * LCAP,), jnp.int32),
            pltpu.VMEM((LANES,), jnp.float32),
            pltpu.VMEM((RPT * LANES,), jnp.float32),
            pltpu.VMEM((RPT * LANES,), jnp.float32),
            pltpu.VMEM((RPT * LANES,), jnp.float32),
            pltpu.VMEM((RPT * LANES,), jnp.float32),
            pltpu.VMEM((RPT * LANES,), jnp.int32),
            pltpu.VMEM((RPT * K,), jnp.float32),
            pltpu.VMEM((RPT * K,), jnp.int32),
            pltpu.VMEM((RPT,), jnp.float32),
            pltpu.VMEM((RPT,), jnp.float32),
            pltpu.SemaphoreType.DMA,
            pltpu.SemaphoreType.DMA,
        ],
    )
    def sc_kernel(scores_hbm, tail_hbm, tok_hbm, ov_hbm, oc_hbm, om_hbm,
                  os_hbm, bufA, bufB, bufS, bufU, tokbuf, candv, candc, srt,
                  strun, slmax, sm0, ssum, scnt, t8v, t8c, m8, s8,
                  semA, semB):
        wid = lax.axis_index("s") * NC + lax.axis_index("c")
        r0 = wid * RPT
        lane = lax.iota(jnp.int32, LANES)

        pltpu.sync_copy(tok_hbm.at[pl.ds(r0, RPT), :], tokbuf)

        def cinit(i, _):
            candv[pl.ds(i * LANES, LANES)] = jnp.full((LANES,), NEG, jnp.float32)
            return 0
        lax.fori_loop(0, RPT * LCAP, cinit, 0)

        def start(c, dst, sem, w):
            pltpu.async_copy(
                scores_hbm.at[pl.ds(r0, RPT), pl.ds(c * BC, w)], dst, sem)

        def wait(dst, sem, w):
            pltpu.make_async_copy(
                scores_hbm.at[pl.ds(0, RPT), pl.ds(0, w)], dst, sem).wait()

        def penalty(sbuf, r, c0, w):
            for t in range(TOKVECS):
                tok = tokbuf[r, pl.ds(t * LANES, LANES)]
                loc = tok - c0
                msk = (loc >= 0) & (loc < w)
                locc = jnp.where(msk, loc, 0)
                rsp = jnp.broadcast_to(r, (LANES,))
                g = plsc.load_gather(sbuf, [rsp, locc], mask=msk)
                pen = jnp.where(g < 0.0, g * REP, g * (1.0 / REP))
                plsc.store_scatter(sbuf, [rsp, locc], pen, mask=msk)

        def splat8th(lmax):
            srt[...] = plsc.sort_key_val(lmax, lmax, descending=True)[0]
            return jnp.broadcast_to(_load1(srt, K - 1), (LANES,))

        def process(sbuf, c0, w, do_penalty=True):
            nv = w // LANES

            def rbody(r, _):
                if do_penalty:
                    penalty(sbuf, r, c0, w)
                rb = r * LANES
                ringb = lane * LCAP + r * (LANES * LCAP)
                t_run = strun[pl.ds(rb, LANES)]
                m0 = sm0[pl.ds(rb, LANES)]

                def sweep(i, car):
                    lmax, sacc, cnt = car
                    x = sbuf[r, pl.ds(i * LANES, LANES)]
                    lmax = jnp.maximum(lmax, x)
                    sacc = sacc + jnp.exp(x - m0)
                    msk = x >= t_run
                    idx = ringb + jnp.minimum(cnt, LCAP - 1)
                    plsc.store_scatter(candv, [idx], x, mask=msk)
                    plsc.store_scatter(candc, [idx], lane + (c0 + i * LANES),
                                       mask=msk)
                    return lmax, sacc, cnt + msk.astype(jnp.int32)

                lmax, sacc, cnt = lax.fori_loop(
                    0, nv, sweep,
                    (slmax[pl.ds(rb, LANES)], ssum[pl.ds(rb, LANES)],
                     scnt[pl.ds(rb, LANES)]), unroll=8)
                slmax[pl.ds(rb, LANES)] = lmax
                ssum[pl.ds(rb, LANES)] = sacc
                scnt[pl.ds(rb, LANES)] = cnt
                strun[pl.ds(rb, LANES)] = splat8th(lmax)
                return 0

            lax.fori_loop(0, RPT, rbody, 0)

        def process0(sbuf, w):
            nv = w // LANES

            def rbody(r, _):
                penalty(sbuf, r, 0, w)
                rb = r * LANES

                def bmax(i, mv):
                    return jnp.maximum(mv, sbuf[r, pl.ds(i * LANES, LANES)])
                lmax = lax.fori_loop(0, nv, bmax,
                                     jnp.full((LANES,), NEG, jnp.float32),
                                     unroll=8)
                sm0[pl.ds(rb, LANES)] = jnp.broadcast_to(jnp.max(lmax),
                                                         (LANES,))
                strun[pl.ds(rb, LANES)] = splat8th(lmax)
                slmax[pl.ds(rb, LANES)] = jnp.full((LANES,), NEG, jnp.float32)
                ssum[pl.ds(rb, LANES)] = jnp.zeros((LANES,), jnp.float32)
                scnt[pl.ds(rb, LANES)] = jnp.zeros((LANES,), jnp.int32)
                return 0

            lax.fori_loop(0, RPT, rbody, 0)
            process(sbuf, 0, w, do_penalty=False)

        start(0, bufA, semA, BC)
        start(1, bufB, semB, BC)
        wait(bufA, semA, BC)
        process0(bufA, BC)
        start(2, bufA, semA, BC)

        def group(g, _):
            c = 2 * g + 1
            wait(bufB, semB, BC)
            process(bufB, c * BC, BC)
            start(c + 2, bufB, semB, BC)
            wait(bufA, semA, BC)
            process(bufA, (c + 1) * BC, BC)
            lax.cond(c + 3 <= NFULL - 1,
                     lambda: start(c + 3, bufA, semA, BC), lambda: None)
            return 0

        lax.fori_loop(0, (NFULL - 2) // 2, group, 0)
        wait(bufB, semB, BC)
        pltpu.async_copy(
            scores_hbm.at[pl.ds(r0, RPT), pl.ds(NFULL * BC, SLABS)],
            bufS, semA)
        process(bufB, (NFULL - 1) * BC, BC)
        pltpu.make_async_copy(
            scores_hbm.at[pl.ds(0, RPT), pl.ds(0, SLABS)], bufS, semA).wait()
        pltpu.async_copy(tail_hbm.at[pl.ds(r0, RPT), :], bufU, semB)
        process(bufS, NFULL * BC, SLABS)
        pltpu.make_async_copy(
            tail_hbm.at[pl.ds(0, RPT), :], bufU, semB).wait()
        process(bufU, NFULL * BC + SLABS, VTAIL)

        def drow(r, _):
            rbase = r * (LANES * LCAP)
            rb = r * LANES

            def dstep(j, _2):
                def body(i, car):
                    bv, bc, bp = car
                    x = candv[pl.ds(rbase + i * LANES, LANES)]
                    c = candc[pl.ds(rbase + i * LANES, LANES)]
                    upd = (x > bv) | ((x == bv) & (c < bc))
                    return (jnp.where(upd, x, bv), jnp.where(upd, c, bc),
                            jnp.where(upd, _iota16(i * LANES), bp))
                bv, bc, bp = lax.fori_loop(
                    0, LCAP, body,
                    (jnp.full((LANES,), NEG, jnp.float32),
                     jnp.full((LANES,), BIGI, jnp.int32),
                     jnp.full((LANES,), BIGI, jnp.int32)), unroll=4)
                m = jnp.max(bv)
                hit = bv >= m
                col = jnp.min(jnp.where(hit, bc, BIGI))
                pos = jnp.min(jnp.where(hit & (bc == col), bp, BIGI))
                _store1(candv, rbase + pos, jnp.float32(NEG))
                _store1(t8v, r * K + j, m)
                _store1(t8c, r * K + j, col)
                return 0
            lax.fori_loop(0, K, dstep, 0)

            m_row = jnp.max(slmax[pl.ds(rb, LANES)])
            _store1(m8, r, m_row)
            scale = jnp.exp(sm0[pl.ds(rb, LANES)]
                            - jnp.broadcast_to(m_row, (LANES,)))
            _store1(s8, r, jnp.sum(ssum[pl.ds(rb, LANES)] * scale))
            return 0
        lax.fori_loop(0, RPT, drow, 0)

        pltpu.sync_copy(t8v, ov_hbm.at[pl.ds(r0 * K, RPT * K)])
        pltpu.sync_copy(t8c, oc_hbm.at[pl.ds(r0 * K, RPT * K)])
        pltpu.sync_copy(m8, om_hbm.at[pl.ds(r0, RPT)])
        pltpu.sync_copy(s8, os_hbm.at[pl.ds(r0, RPT)])

    return sc_kernel(scores, scores_tail, tokens_p)


def _tc_merge_body(v_ref, c_ref, m_ref, s_ref, b_ref,
                   ks_ref, kt_ref, kb_ref, ri_ref):
    x = v_ref[:]
    cols = c_ref[:]
    logp = x - m_ref[:] - jnp.log(s_ref[:]) + b_ref[:]
    pos = lax.broadcasted_iota(jnp.int32, (B, NB * K), 1)

    remaining = logp
    nb_seen = jnp.zeros((B, 1), jnp.int32)
    ks = [jnp.zeros((B, 1), jnp.float32) for _ in range(NB)]
    kt = [jnp.zeros((B, 1), jnp.int32) for _ in range(NB)]
    kb = [jnp.zeros((B, 1), jnp.int32) for _ in range(NB)]
    for _ in range(K):
        mx = jnp.max(remaining, axis=1, keepdims=True)
        p = jnp.min(jnp.where(remaining >= mx, pos, BIGI), axis=1, keepdims=True)
        hit = pos == p
        tok = jnp.sum(jnp.where(hit, cols, 0), axis=1, keepdims=True)
        beam = p // K
        remaining = jnp.where(hit, NEG, remaining)
        not_eos = (tok != EOS).astype(jnp.int32)
        rank = nb_seen + not_eos
        for q in range(NB):
            keep = (not_eos == 1) & (rank == q + 1)
            ks[q] = jnp.where(keep, mx, ks[q])
            kt[q] = jnp.where(keep, tok, kt[q])
            kb[q] = jnp.where(keep, beam, kb[q])
        nb_seen = nb_seen + not_eos

    ks_ref[:] = jnp.concatenate(ks, axis=1)
    kt_ref[:] = jnp.concatenate(kt, axis=1)
    kbm = jnp.concatenate(kb, axis=1)
    kb_ref[:] = kbm
    ri_ref[:] = kbm + jnp.arange(B, dtype=jnp.int32)[:, None] * NB


def _tc_merge(vals, cols, m, s, beam_scores):
    v32 = vals.reshape(B, NB * K)
    c32 = cols.reshape(B, NB * K)
    m32 = jnp.repeat(m.reshape(B, NB), K, axis=1)
    s32 = jnp.repeat(s.reshape(B, NB), K, axis=1)
    b32 = jnp.repeat(beam_scores.reshape(B, NB), K, axis=1)
    ks, kt, kb, ri = pl.pallas_call(
        _tc_merge_body,
        out_shape=(
            jax.ShapeDtypeStruct((B, NB), jnp.float32),
            jax.ShapeDtypeStruct((B, NB), jnp.int32),
            jax.ShapeDtypeStruct((B, NB), jnp.int32),
            jax.ShapeDtypeStruct((B, NB), jnp.int32),
        ),
    )(v32, c32, m32, s32, b32)
    return ks, kt, kb, ri.reshape(-1)


def kernel(scores, beam_scores, token_ids):
    tokens_p = jnp.pad(token_ids, ((0, 0), (0, TOKP - token_ids.shape[1])),
                       constant_values=BIGI)
    scores_tail = lax.slice(scores, (0, NFULL * BC + SLABS), (BN, V))
    vals, cols, m, s = _sc_stage(scores, scores_tail, tokens_p)
    return _tc_merge(vals, cols, m, s, beam_scores)
